# baseline (speedup 1.0000x reference)
import functools

import jax
import jax.numpy as jnp
from jax import lax
from jax.experimental import pallas as pl
from jax.experimental.pallas import tpu as pltpu
from jax.experimental.pallas import tpu_sc as plsc

N = 10000
DIN = 128
DH = 32
NCLS = 2
E = 320000

NC = 2
NS = 16
NW = NC * NS
CHUNK = 128
NCH = 80
EPT = NCH * CHUNK
EPAD = NW * EPT
NBUF = 16
LOOK = 8
NBUF2 = 8
LOOK2 = 4
RG = 160
BM = 1280

NDEG = 10240
DEG_SL = NDEG // NS
NACC = 10240
ROW_SL = NACC // NS

_sc_mesh = plsc.VectorSubcoreMesh(
    core_axis_name="c", subcore_axis_name="s", num_cores=NC, num_subcores=NS
)
_sc_params = pltpu.CompilerParams(use_tc_tiling_on_sc=False)


@functools.partial(
    pl.kernel,
    out_type=jax.ShapeDtypeStruct((NC, NDEG), jnp.float32),
    mesh=_sc_mesh,
    compiler_params=_sc_params,
    scratch_types=[
        pltpu.VMEM_SHARED((NDEG,), jnp.float32),
        pltpu.VMEM((NCH, CHUNK), jnp.int32),
        pltpu.VMEM((CHUNK,), jnp.float32),
        pltpu.SemaphoreType.DMA,
    ],
)
def _deg_kernel(dst_hbm, zeros1_hbm, ones_hbm, deg_out, deg_sp, dst_all, ones_v, sem):
    c = lax.axis_index("c")
    s = lax.axis_index("s")
    wid = c * NS + s
    pltpu.sync_copy(
        zeros1_hbm.at[pl.ds(s * DEG_SL, DEG_SL)],
        deg_sp.at[pl.ds(s * DEG_SL, DEG_SL)],
    )
    pltpu.sync_copy(ones_hbm, ones_v)
    pltpu.sync_copy(dst_hbm.at[wid], dst_all)
    plsc.subcore_barrier()

    def body(o, carry):
        for b in range(8):
            pltpu.async_copy(ones_v, deg_sp.at[dst_all.at[o * 8 + b]], sem, add=True)
        for b in range(8):
            pltpu.make_async_copy(ones_hbm, ones_v, sem).wait()
        return carry

    lax.fori_loop(0, NCH // 8, body, 0)
    plsc.subcore_barrier()
    pltpu.sync_copy(
        deg_sp.at[pl.ds(s * DEG_SL, DEG_SL)],
        deg_out.at[c].at[pl.ds(s * DEG_SL, DEG_SL)],
    )


@functools.partial(
    pl.kernel,
    out_type=jax.ShapeDtypeStruct((NC, NACC, DH), jnp.float32),
    mesh=_sc_mesh,
    compiler_params=_sc_params,
    scratch_types=[
        pltpu.VMEM_SHARED((NACC, DH), jnp.float32),
        pltpu.VMEM((NCH, CHUNK), jnp.int32),
        pltpu.VMEM((NCH, CHUNK), jnp.int32),
        pltpu.VMEM((NBUF, CHUNK, DH), jnp.float32),
        pltpu.SemaphoreType.DMA((NBUF,)),
    ],
)
def _msg_kernel(hs_hbm, src_hbm, dst_hbm, zeros2_hbm, acc_out,
                acc_sp, src_all, dst_all, rows, gsem):
    c = lax.axis_index("c")
    s = lax.axis_index("s")
    wid = c * NS + s
    pltpu.sync_copy(
        zeros2_hbm.at[pl.ds(s * ROW_SL, ROW_SL)],
        acc_sp.at[pl.ds(s * ROW_SL, ROW_SL)],
    )
    pltpu.sync_copy(src_hbm.at[wid], src_all)
    pltpu.sync_copy(dst_hbm.at[wid], dst_all)
    plsc.subcore_barrier()

    for k in range(LOOK):
        pltpu.async_copy(hs_hbm.at[src_all.at[k]], rows.at[k], gsem.at[k])

    def body(o, carry):
        for b in range(NBUF):
            ch = o * NBUF + b
            bf = (b + LOOK) % NBUF

            @pl.when(ch + LOOK < NCH)
            def _():
                pltpu.async_copy(
                    hs_hbm.at[src_all.at[ch + LOOK]], rows.at[bf], gsem.at[bf]
                )

            pltpu.make_async_copy(
                hs_hbm.at[pl.ds(0, CHUNK)], rows.at[b], gsem.at[b]
            ).wait()
            pltpu.sync_copy(rows.at[b], acc_sp.at[dst_all.at[ch]], add=True)
        return carry

    lax.fori_loop(0, NCH // NBUF, body, 0)
    plsc.subcore_barrier()
    pltpu.sync_copy(
        acc_sp.at[pl.ds(s * ROW_SL, ROW_SL)],
        acc_out.at[c].at[pl.ds(s * ROW_SL, ROW_SL)],
    )


@functools.partial(
    pl.kernel,
    out_type=(
        jax.ShapeDtypeStruct((NC, NACC, DH), jnp.float32),
        jax.ShapeDtypeStruct((NACC, DH), jnp.float32),
    ),
    mesh=_sc_mesh,
    compiler_params=_sc_params,
    scratch_types=[
        pltpu.VMEM_SHARED((NACC, DH), jnp.float32),
        pltpu.VMEM_SHARED((NACC, DH), jnp.float32),
        pltpu.VMEM((RG, DH), jnp.float32),
        pltpu.VMEM((RG, DH), jnp.float32),
        pltpu.VMEM((RG, DH), jnp.float32),
        pltpu.VMEM((RG, DH), jnp.float32),
        pltpu.VMEM((RG, DH), jnp.float32),
        pltpu.VMEM((DH,), jnp.float32),
        pltpu.VMEM((NCH, CHUNK), jnp.int32),
        pltpu.VMEM((NCH, CHUNK), jnp.int32),
        pltpu.VMEM((NBUF2, CHUNK, DH), jnp.float32),
        pltpu.SemaphoreType.DMA((NBUF2,)),
    ],
)
def _msg2_kernel(acc1_hbm, hs1_hbm, drep_hbm, b1_hbm, src_hbm, dst_hbm,
                 zeros2_hbm, acc2_out, hs2_out,
                 hs2_sp, acc2_sp, a0v, a1v, h1v, drv, h2v, b1v,
                 src_all, dst_all, rows, gsem):
    c = lax.axis_index("c")
    s = lax.axis_index("s")
    wid = c * NS + s
    pltpu.sync_copy(
        zeros2_hbm.at[pl.ds(s * ROW_SL, ROW_SL)],
        acc2_sp.at[pl.ds(s * ROW_SL, ROW_SL)],
    )
    pltpu.sync_copy(src_hbm.at[wid], src_all)
    pltpu.sync_copy(dst_hbm.at[wid], dst_all)
    pltpu.sync_copy(b1_hbm, b1v)

    def group(g, carry):
        base = s * ROW_SL + g * RG
        pltpu.sync_copy(acc1_hbm.at[0].at[pl.ds(base, RG)], a0v)
        pltpu.sync_copy(acc1_hbm.at[1].at[pl.ds(base, RG)], a1v)
        pltpu.sync_copy(hs1_hbm.at[pl.ds(base, RG)], h1v)
        pltpu.sync_copy(drep_hbm.at[pl.ds(base, RG)], drv)

        def row(r, carry2):
            for h in range(DH // 16):
                sl = pl.ds(h * 16, 16)
                d = drv[r, sl]
                v = (a0v[r, sl] + a1v[r, sl] + h1v[r, sl]) * d
                v = jnp.maximum(v + b1v[pl.ds(h * 16, 16)], 0.0)
                h2v[r, sl] = v * d
            return carry2

        lax.fori_loop(0, RG, row, 0)
        pltpu.sync_copy(h2v, hs2_sp.at[pl.ds(base, RG)])
        return carry

    lax.fori_loop(0, ROW_SL // RG, group, 0)
    plsc.subcore_barrier()

    for k in range(LOOK2):
        pltpu.async_copy(hs2_sp.at[src_all.at[k]], rows.at[k], gsem.at[k])

    def body(o, carry):
        for b in range(NBUF2):
            ch = o * NBUF2 + b
            bf = (b + LOOK2) % NBUF2

            @pl.when(ch + LOOK2 < NCH)
            def _():
                pltpu.async_copy(
                    hs2_sp.at[src_all.at[ch + LOOK2]], rows.at[bf], gsem.at[bf]
                )

            pltpu.make_async_copy(
                hs1_hbm.at[pl.ds(0, CHUNK)], rows.at[b], gsem.at[b]
            ).wait()
            pltpu.sync_copy(rows.at[b], acc2_sp.at[dst_all.at[ch]], add=True)
        return carry

    lax.fori_loop(0, NCH // NBUF2, body, 0)
    plsc.subcore_barrier()
    pltpu.sync_copy(
        acc2_sp.at[pl.ds(s * ROW_SL, ROW_SL)],
        acc2_out.at[c].at[pl.ds(s * ROW_SL, ROW_SL)],
    )

    @pl.when(c == 0)
    def _():
        pltpu.sync_copy(
            hs2_sp.at[pl.ds(s * ROW_SL, ROW_SL)],
            hs2_out.at[pl.ds(s * ROW_SL, ROW_SL)],
        )


def _layer1_body(x_ref, w1_ref, degcol_ref, hs1_ref, drep_ref):
    d = lax.rsqrt(degcol_ref[...])
    h = jnp.dot(x_ref[...], w1_ref[...], preferred_element_type=jnp.float32)
    hs1_ref[...] = h * d
    drep_ref[...] = d * jnp.ones((1, DH), jnp.float32)


def _final_body(acc_ref, hs2_ref, drep_ref, w2_ref, b2_ref, out_ref):
    p2 = (acc_ref[0] + acc_ref[1] + hs2_ref[...]) * drep_ref[...]
    logits = (
        jnp.dot(p2, w2_ref[...], preferred_element_type=jnp.float32)
        + b2_ref[...]
    )
    l0 = logits[:, 0:1]
    l1 = logits[:, 1:2]
    mx = jnp.maximum(l0, l1)
    lse = mx + jnp.log(jnp.exp(l0 - mx) + jnp.exp(l1 - mx))
    out_ref[...] = logits - lse


def kernel(x, edge_index, W1, b1, W2, b2):
    npad = EPAD - E
    pad_ar = jnp.arange(npad, dtype=jnp.int32)
    pad_src = (pad_ar * 13) % N
    pad_dst = N + pad_ar % (NACC - N)
    src = jnp.concatenate([edge_index[0], pad_src]).reshape(NW, NCH, CHUNK)
    dst = jnp.concatenate([edge_index[1], pad_dst]).reshape(NW, NCH, CHUNK)
    zeros1 = jnp.zeros((NDEG,), jnp.float32)
    zeros2 = jnp.zeros((NACC, DH), jnp.float32)
    ones = jnp.ones((CHUNK,), jnp.float32)

    degp = _deg_kernel(dst, zeros1, ones)
    degcol = (degp[0] + degp[1] + 1.0)[:, None]

    hs1, drep = pl.pallas_call(
        _layer1_body,
        grid=(NACC // BM,),
        in_specs=[
            pl.BlockSpec((BM, DIN), lambda i: (i, 0)),
            pl.BlockSpec((DIN, DH), lambda i: (0, 0)),
            pl.BlockSpec((BM, 1), lambda i: (i, 0)),
        ],
        out_specs=[
            pl.BlockSpec((BM, DH), lambda i: (i, 0)),
            pl.BlockSpec((BM, DH), lambda i: (i, 0)),
        ],
        out_shape=[
            jax.ShapeDtypeStruct((NACC, DH), jnp.float32),
            jax.ShapeDtypeStruct((NACC, DH), jnp.float32),
        ],
    )(x, W1, degcol)

    acc1 = _msg_kernel(hs1, src, dst, zeros2)

    acc2, hs2 = _msg2_kernel(acc1, hs1, drep, b1, src, dst, zeros2)

    out = pl.pallas_call(
        _final_body,
        grid=(NACC // BM,),
        in_specs=[
            pl.BlockSpec((NC, BM, DH), lambda i: (0, i, 0)),
            pl.BlockSpec((BM, DH), lambda i: (i, 0)),
            pl.BlockSpec((BM, DH), lambda i: (i, 0)),
            pl.BlockSpec((DH, NCLS), lambda i: (0, 0)),
            pl.BlockSpec((1, NCLS), lambda i: (0, 0)),
        ],
        out_specs=pl.BlockSpec((BM, NCLS), lambda i: (i, 0)),
        out_shape=jax.ShapeDtypeStruct((N, NCLS), jnp.float32),
    )(acc2, hs2, drep, W2, b2[None, :])
    return out
